# Initial kernel scaffold; baseline (speedup 1.0000x reference)
import jax
import jax.numpy as jnp
from jax.experimental import pallas as pl


def kernel(x, edge_index, W1l, b1l, W1r, g1, be1, W2l, b2l, W2r, g2, be2):
    raise NotImplementedError("write your pallas kernel here")


# speedup vs baseline: 5.9438x; 5.9438x over previous
import functools

import jax
import jax.numpy as jnp
from jax import lax
from jax.experimental import pallas as pl
from jax.experimental.pallas import tpu as pltpu
from jax.experimental.pallas import tpu_sc as plsc

_NC = 2
_NS = 16
_L = 16
_CH = 128


@functools.lru_cache(maxsize=None)
def _make_sc_seg_sum(n, n_pad, dh, n_chunks, with_deg):
    rows_per_tile = n_pad // _NS
    full = rows_per_tile // _CH
    rem = rows_per_tile % _CH

    mesh = plsc.VectorSubcoreMesh(core_axis_name="c", subcore_axis_name="s")
    out_type = [jax.ShapeDtypeStruct((_NC * n_pad, dh), jnp.float32)]
    if with_deg:
        out_type.append(jax.ShapeDtypeStruct((n_pad, _L), jnp.float32))
    scratch = [
        pltpu.VMEM((n_chunks, _CH), jnp.int32),
        pltpu.VMEM((n_chunks, _CH), jnp.int32),
        pltpu.VMEM((_CH, dh), jnp.float32),
        pltpu.VMEM((_CH, _L), jnp.float32),
        pltpu.VMEM((_CH, _L), jnp.float32),
        pltpu.VMEM_SHARED((n_pad, dh), jnp.float32),
        pltpu.VMEM_SHARED((n_pad, _L), jnp.float32),
        pltpu.SemaphoreType.DMA,
    ]

    def body(xs_hbm, src_hbm, dst_hbm, *rest):
        if with_deg:
            out_s, out_d = rest[0], rest[1]
            rest = rest[2:]
        else:
            out_s, out_d = rest[0], None
            rest = rest[1:]
        src_v, dst_v, rows_v, z16_v, e0_v, sum_sp, deg_sp, gsem = rest

        c = lax.axis_index("c")
        s = lax.axis_index("s")
        wid = c * _NS + s

        pltpu.sync_copy(src_hbm.at[wid], src_v)
        pltpu.sync_copy(dst_hbm.at[s], dst_v)

        e0 = jnp.where(lax.iota(jnp.int32, _L) == 0,
                       jnp.float32(1.0), jnp.float32(0.0))
        z = jnp.zeros((_L,), jnp.float32)

        def fill_row(i, carry):
            for k in range(dh // _L):
                rows_v[i, pl.ds(k * _L, _L)] = z
            z16_v[i, pl.ds(0, _L)] = z
            e0_v[i, pl.ds(0, _L)] = e0
            return carry

        lax.fori_loop(0, _CH, fill_row, 0)

        r0 = s * rows_per_tile
        for b in range(full):
            pltpu.sync_copy(rows_v, sum_sp.at[pl.ds(r0 + b * _CH, _CH)])
            if with_deg:
                pltpu.sync_copy(z16_v, deg_sp.at[pl.ds(r0 + b * _CH, _CH)])
        if rem:
            pltpu.sync_copy(rows_v.at[pl.ds(0, rem)],
                            sum_sp.at[pl.ds(r0 + full * _CH, rem)])
            if with_deg:
                pltpu.sync_copy(z16_v.at[pl.ds(0, rem)],
                                deg_sp.at[pl.ds(r0 + full * _CH, rem)])
        plsc.subcore_barrier()

        def chunk(j, carry):
            pltpu.async_copy(xs_hbm.at[src_v.at[j]], rows_v, gsem).wait()
            pltpu.sync_copy(rows_v, sum_sp.at[dst_v.at[j]], add=True)
            if with_deg:
                @pl.when(c == 0)
                def _():
                    pltpu.sync_copy(e0_v, deg_sp.at[dst_v.at[j]], add=True)
            return carry

        lax.fori_loop(0, n_chunks, chunk, 0)
        plsc.subcore_barrier()

        pltpu.sync_copy(sum_sp.at[pl.ds(r0, rows_per_tile)],
                        out_s.at[pl.ds(c * n_pad + r0, rows_per_tile)])
        if with_deg:
            @pl.when(c == 0)
            def _():
                pltpu.sync_copy(deg_sp.at[pl.ds(r0, rows_per_tile)],
                                out_d.at[pl.ds(r0, rows_per_tile)])

    return pl.kernel(body, out_type=out_type, mesh=mesh,
                     scratch_types=scratch,
                     compiler_params=pltpu.CompilerParams(
                         use_tc_tiling_on_sc=False))


@functools.lru_cache(maxsize=None)
def _make_tc_dense(n, d, split_in, split_out, apply_relu):
    dh = d // 2

    def body(s0, s1, dg, xin, wl, bl, wr, g, be, out):
        summed = jnp.concatenate([s0[...], s1[...]], axis=1)
        deg = dg[...][:, 0:1]
        agg = summed / jnp.maximum(deg, 1.0)
        xf = xin[...]
        if split_in:
            xf = jnp.concatenate([xf[0:n], xf[n:2 * n]], axis=1)
        h = (lax.dot_general(agg, wl[...], (((1,), (1,)), ((), ())),
                             preferred_element_type=jnp.float32)
             + bl[...]
             + lax.dot_general(xf, wr[...], (((1,), (1,)), ((), ())),
                               preferred_element_type=jnp.float32))
        mu = jnp.mean(h, axis=0, keepdims=True)
        var = jnp.mean((h - mu) * (h - mu), axis=0, keepdims=True)
        y = (h - mu) * lax.rsqrt(var + 1e-5) * g[...] + be[...]
        if apply_relu:
            y = jnp.maximum(y, 0.0)
        if split_out:
            out[0:n, :] = y[:, 0:dh]
            out[n:2 * n, :] = y[:, dh:d]
        else:
            out[...] = y

    out_shape = jax.ShapeDtypeStruct((2 * n, dh) if split_out else (n, d),
                                     jnp.float32)
    return pl.pallas_call(body, out_shape=out_shape)


def kernel(x, edge_index, W1l, b1l, W1r, g1, be1, W2l, b2l, W2r, g2, be2):
    n, d = x.shape
    dh = d // 2
    e = edge_index.shape[1]
    n_pad = ((n + 1 + 127) // 128) * 128
    n_chunks = -(-e // (_NS * _CH))
    e_pad = n_chunks * _NS * _CH

    src = edge_index[0].astype(jnp.int32)
    dst = edge_index[1].astype(jnp.int32)
    pad = e_pad - e
    if pad:
        src = jnp.concatenate([src, jnp.zeros((pad,), jnp.int32)])
        dst = jnp.concatenate([dst, jnp.full((pad,), n, jnp.int32)])
    src2 = jnp.stack([src, src + n]).reshape(_NC * _NS, n_chunks, _CH)
    dst = dst.reshape(_NS, n_chunks, _CH)
    xs = jnp.concatenate([x[:, :dh], x[:, dh:]], axis=0)

    sc_l1 = _make_sc_seg_sum(n, n_pad, dh, n_chunks, True)
    sc_l2 = _make_sc_seg_sum(n, n_pad, dh, n_chunks, False)
    tc_l1 = _make_tc_dense(n, d, False, True, True)
    tc_l2 = _make_tc_dense(n, d, True, False, False)

    s_all, deg = sc_l1(xs, src2, dst)
    h1s = tc_l1(s_all[:n], s_all[n_pad:n_pad + n], deg[:n], x,
                W1l, b1l.reshape(1, d), W1r, g1.reshape(1, d),
                be1.reshape(1, d))

    out2 = sc_l2(h1s, src2, dst)
    s_all2 = out2[0] if isinstance(out2, (tuple, list)) else out2
    h2 = tc_l2(s_all2[:n], s_all2[n_pad:n_pad + n], deg[:n], h1s,
               W2l, b2l.reshape(1, d), W2r, g2.reshape(1, d),
               be2.reshape(1, d))
    return h2
